# Initial kernel scaffold; baseline (speedup 1.0000x reference)
import jax
import jax.numpy as jnp
from jax.experimental import pallas as pl


def kernel(x, edge_index, edge_weight, W):
    raise NotImplementedError("write your pallas kernel here")


# speedup vs baseline: 6.2804x; 6.2804x over previous
import functools

import jax
import jax.numpy as jnp
from jax import lax
from jax.experimental import pallas as pl
from jax.experimental.pallas import tpu as pltpu
from jax.experimental.pallas import tpu_sc as plsc

N = 10000
E = 320000
D = 128

NC = 2
NS = 16
NW = NC * NS
EPT = E // NW
CH = 80
NB = 5
BC = 25
RPT = 624


def _mm_body(x_ref, w_ref, o_ref):
    o_ref[...] = jnp.dot(x_ref[...], w_ref[...],
                         preferred_element_type=jnp.float32)


def _matmul(x, W):
    return pl.pallas_call(
        _mm_body,
        grid=(10,),
        in_specs=[
            pl.BlockSpec((N // 10, D), lambda i: (i, 0)),
            pl.BlockSpec((D, D), lambda i: (0, 0)),
        ],
        out_specs=pl.BlockSpec((N // 10, D), lambda i: (i, 0)),
        out_shape=jax.ShapeDtypeStruct((N, D), jnp.float32),
    )(x, W)


_sc_mesh = plsc.VectorSubcoreMesh(core_axis_name="c", subcore_axis_name="s")


@functools.partial(
    pl.kernel,
    out_type=jax.ShapeDtypeStruct((NC, N, D), jnp.float32),
    mesh=_sc_mesh,
    scratch_types=[
        pltpu.VMEM((BC, CH), jnp.int32),
        pltpu.VMEM((BC, CH), jnp.int32),
        pltpu.VMEM((BC, CH), jnp.float32),
        pltpu.VMEM((CH, D), jnp.float32),
        pltpu.VMEM_SHARED((N, D), jnp.float32),
        pltpu.SemaphoreType.DMA,
    ],
)
def _sc_spmm(h_hbm, src_hbm, dst_hbm, w_hbm, out_hbm,
             src_v, dst_v, w_v, rows_v, agg_sh, sem):
    cid = lax.axis_index("c")
    sid = lax.axis_index("s")
    wid = cid * NS + sid

    zeros16 = jnp.zeros((16,), jnp.float32)

    def _z(i, _):
        r = i // (D // 16)
        v = i % (D // 16)
        rows_v[r, pl.ds(v * 16, 16)] = zeros16
        return 0

    lax.fori_loop(0, CH * (D // 16), _z, 0)
    for j in range(8):
        pltpu.sync_copy(rows_v, agg_sh.at[pl.ds(sid * RPT + j * CH, CH)])
    plsc.subcore_barrier()

    def _block(b, _):
        pltpu.sync_copy(src_hbm.at[wid, b], src_v)
        pltpu.sync_copy(dst_hbm.at[wid, b], dst_v)
        pltpu.sync_copy(w_hbm.at[wid, b], w_v)

        def _chunk(i, _):
            pltpu.async_copy(h_hbm.at[src_v.at[i]], rows_v, sem).wait()

            def _group(g, _):
                wv = w_v[i, pl.ds(g * 16, 16)]
                for j in range(16):
                    w = wv[j]
                    for v in range(D // 16):
                        e = g * 16 + j
                        rows_v[e, pl.ds(v * 16, 16)] = (
                            rows_v[e, pl.ds(v * 16, 16)] * w)
                return 0

            lax.fori_loop(0, CH // 16, _group, 0)
            pltpu.sync_copy(rows_v, agg_sh.at[dst_v.at[i]], add=True)
            return 0

        lax.fori_loop(0, BC, _chunk, 0)
        return 0

    lax.fori_loop(0, NB, _block, 0)
    plsc.subcore_barrier()

    pltpu.sync_copy(agg_sh.at[pl.ds(sid * RPT, RPT)],
                    out_hbm.at[cid, pl.ds(sid * RPT, RPT)])

    @pl.when(sid == NS - 1)
    def _tail():
        pltpu.sync_copy(agg_sh.at[pl.ds(NS * RPT, N - NS * RPT)],
                        out_hbm.at[cid, pl.ds(NS * RPT, N - NS * RPT)])


def _comb_body(a_ref, b_ref, o_ref):
    o_ref[...] = jnp.maximum(a_ref[...] + b_ref[...], 0.0)


def _combine(p0, p1):
    return pl.pallas_call(
        _comb_body,
        grid=(10,),
        in_specs=[
            pl.BlockSpec((N // 10, D), lambda i: (i, 0)),
            pl.BlockSpec((N // 10, D), lambda i: (i, 0)),
        ],
        out_specs=pl.BlockSpec((N // 10, D), lambda i: (i, 0)),
        out_shape=jax.ShapeDtypeStruct((N, D), jnp.float32),
    )(p0, p1)


def kernel(x, edge_index, edge_weight, W):
    h = _matmul(x, W)
    src = edge_index[0].astype(jnp.int32).reshape(NW, NB, BC, CH)
    dst = edge_index[1].astype(jnp.int32).reshape(NW, NB, BC, CH)
    w = edge_weight.reshape(NW, NB, BC, CH)
    partials = _sc_spmm(h, src, dst, w)
    return _combine(partials[0], partials[1])
